# baseline (speedup 1.0000x reference)
import functools

import jax
import jax.numpy as jnp
from jax import lax
from jax.experimental import pallas as pl
from jax.experimental.pallas import tpu as pltpu
from jax.experimental.pallas import tpu_sc as plsc

_NJ = 200
_IB = 128
_D = 64


def _body(num_cores, tok_hbm, w_hbm, out_hbm, idx_v, rows_v, stage_v, sem_i, sem_g, sem_o):
    wid = lax.axis_index("s") * num_cores + lax.axis_index("c")
    i_base = wid * _IB

    def idx_start(b, bslot):
        pltpu.async_copy(
            tok_hbm.at[pl.ds(b * 8, 8), pl.ds(i_base, _IB)], idx_v.at[bslot], sem_i
        )

    def idx_wait():
        pltpu.make_async_copy(
            tok_hbm.at[pl.ds(0, 8), pl.ds(0, _IB)], idx_v.at[0], sem_i
        ).wait()

    def gather_start(j, slot, bslot, r):
        pltpu.async_copy(
            w_hbm.at[idx_v.at[bslot, r]],
            rows_v.at[pl.ds(slot * _IB, _IB)],
            sem_g,
        )

    def gather_wait(slot):
        pltpu.make_async_copy(
            w_hbm.at[pl.ds(0, _IB)],
            rows_v.at[pl.ds(slot * _IB, _IB)],
            sem_g,
        ).wait()

    def out_start(j, slot):
        pltpu.async_copy(
            stage_v.at[pl.ds(slot * _D, _D)],
            out_hbm.at[j, :, pl.ds(i_base, _IB)],
            sem_o,
        )

    def out_wait(slot):
        pltpu.make_async_copy(
            stage_v.at[pl.ds(slot * _D, _D)],
            out_hbm.at[0, :, pl.ds(i_base, _IB)],
            sem_o,
        ).wait()

    def transpose(slot):
        base = jax.lax.iota(jnp.int32, 16) + slot * _IB
        for c in range(8):
            rvec = base + 16 * c
            for f in range(_D):
                cvec = jnp.full((16,), f, jnp.int32)
                vals = plsc.load_gather(rows_v, [rvec, cvec])
                stage_v[slot * _D + f, pl.ds(16 * c, 16)] = vals

    idx_start(0, 0)
    idx_wait()
    gather_start(0, 0, 0, 0)

    def loop_body(j, carry):
        slot = lax.rem(j, 2)
        nslot = lax.rem(j + 1, 2)
        gather_wait(slot)

        jn = j + 1

        @pl.when(lax.rem(jn, 8) == 0)
        def _():
            idx_wait()

        bn = lax.div(jn, 8)
        gather_start(jn, nslot, lax.rem(bn, 2), lax.rem(jn, 8))

        @pl.when(jnp.logical_and(lax.rem(j + 2, 8) == 0, j + 2 < _NJ))
        def _():
            bb = lax.div(j + 2, 8)
            idx_start(bb, lax.rem(bb, 2))

        @pl.when(j >= 2)
        def _():
            out_wait(slot)

        transpose(slot)
        out_start(j, slot)
        return carry

    lax.fori_loop(0, _NJ - 1, loop_body, 0)

    last = _NJ - 1
    lslot = last % 2
    gather_wait(lslot)
    out_wait(lslot)
    transpose(lslot)
    out_start(last, lslot)
    out_wait(1 - lslot)
    out_wait(lslot)


def kernel(token_ids, W):
    B, T = token_ids.shape
    D = W.shape[1]
    tok_t = token_ids.T
    w_pad = jnp.pad(W, ((0, 0), (0, 128 - D)))
    info = plsc.get_sparse_core_info()

    mesh = plsc.VectorSubcoreMesh(core_axis_name="c", subcore_axis_name="s")
    kfn = pl.kernel(
        functools.partial(_body, info.num_cores),
        out_type=jax.ShapeDtypeStruct((T, D, B), jnp.float32),
        mesh=mesh,
        scratch_types=[
            pltpu.VMEM((2, 8, _IB), jnp.int32),
            pltpu.VMEM((2 * _IB, 128), jnp.float32),
            pltpu.VMEM((2 * _D, 128), jnp.float32),
            pltpu.SemaphoreType.DMA,
            pltpu.SemaphoreType.DMA,
            pltpu.SemaphoreType.DMA,
        ],
        compiler_params=pltpu.CompilerParams(needs_layout_passes=False),
    )
    out3 = kfn(tok_t, w_pad)
    return jnp.transpose(out3, (2, 0, 1))


# speedup vs baseline: 1.4106x; 1.4106x over previous
import functools

import jax
import jax.numpy as jnp
from jax import lax
from jax.experimental import pallas as pl
from jax.experimental.pallas import tpu as pltpu
from jax.experimental.pallas import tpu_sc as plsc

_NJ = 200
_IB = 128
_D = 64


def _body(num_cores, tok_hbm, w_hbm, out_hbm, idx_v, rows_v, stage_v, sem_i, sem_g, sem_o):
    wid = lax.axis_index("s") * num_cores + lax.axis_index("c")
    i_base = wid * _IB

    def idx_start(b, bslot):
        pltpu.async_copy(
            tok_hbm.at[pl.ds(b * 8, 8), pl.ds(i_base, _IB)], idx_v.at[bslot], sem_i
        )

    def idx_wait():
        pltpu.make_async_copy(
            tok_hbm.at[pl.ds(0, 8), pl.ds(0, _IB)], idx_v.at[0], sem_i
        ).wait()

    def gather_start(j, slot, bslot, r):
        pltpu.async_copy(
            w_hbm.at[idx_v.at[bslot, r]],
            rows_v.at[pl.ds(slot * _IB, _IB)],
            sem_g,
        )

    def gather_wait(slot):
        pltpu.make_async_copy(
            w_hbm.at[pl.ds(0, _IB)],
            rows_v.at[pl.ds(slot * _IB, _IB)],
            sem_g,
        ).wait()

    def out_start(j, slot):
        pltpu.async_copy(
            stage_v.at[pl.ds(slot * _D, _D)],
            out_hbm.at[j, :, pl.ds(i_base, _IB)],
            sem_o,
        )

    def out_wait(slot):
        pltpu.make_async_copy(
            stage_v.at[pl.ds(slot * _D, _D)],
            out_hbm.at[0, :, pl.ds(i_base, _IB)],
            sem_o,
        ).wait()

    def transpose(slot):
        base = jax.lax.iota(jnp.int32, 16) + slot * _IB
        for c in range(8):
            rvec = base + 16 * c
            for f0 in range(0, _D, 8):
                vals = [
                    plsc.load_gather(rows_v, [rvec, jnp.full((16,), f0 + k, jnp.int32)])
                    for k in range(8)
                ]
                for k in range(8):
                    stage_v[slot * _D + f0 + k, pl.ds(16 * c, 16)] = vals[k]

    idx_start(0, 0)
    idx_wait()
    gather_start(0, 0, 0, 0)

    def loop_body(j, carry):
        slot = lax.rem(j, 2)
        nslot = lax.rem(j + 1, 2)
        gather_wait(slot)

        jn = j + 1

        @pl.when(lax.rem(jn, 8) == 0)
        def _():
            idx_wait()

        bn = lax.div(jn, 8)
        gather_start(jn, nslot, lax.rem(bn, 2), lax.rem(jn, 8))

        @pl.when(jnp.logical_and(lax.rem(j + 2, 8) == 0, j + 2 < _NJ))
        def _():
            bb = lax.div(j + 2, 8)
            idx_start(bb, lax.rem(bb, 2))

        @pl.when(j >= 2)
        def _():
            out_wait(slot)

        transpose(slot)
        out_start(j, slot)
        return carry

    lax.fori_loop(0, _NJ - 1, loop_body, 0)

    last = _NJ - 1
    lslot = last % 2
    gather_wait(lslot)
    out_wait(lslot)
    transpose(lslot)
    out_start(last, lslot)
    out_wait(1 - lslot)
    out_wait(lslot)


def kernel(token_ids, W):
    B, T = token_ids.shape
    D = W.shape[1]
    tok_t = token_ids.T
    w_pad = jnp.pad(W, ((0, 0), (0, 128 - D)))
    info = plsc.get_sparse_core_info()

    mesh = plsc.VectorSubcoreMesh(core_axis_name="c", subcore_axis_name="s")
    kfn = pl.kernel(
        functools.partial(_body, info.num_cores),
        out_type=jax.ShapeDtypeStruct((T, D, B), jnp.float32),
        mesh=mesh,
        scratch_types=[
            pltpu.VMEM((2, 8, _IB), jnp.int32),
            pltpu.VMEM((2 * _IB, 128), jnp.float32),
            pltpu.VMEM((2 * _D, 128), jnp.float32),
            pltpu.SemaphoreType.DMA,
            pltpu.SemaphoreType.DMA,
            pltpu.SemaphoreType.DMA,
        ],
        compiler_params=pltpu.CompilerParams(needs_layout_passes=False),
    )
    out3 = kfn(tok_t, w_pad)
    return jnp.transpose(out3, (2, 0, 1))


# speedup vs baseline: 1.4227x; 1.0086x over previous
import functools

import jax
import jax.numpy as jnp
from jax import lax
from jax.experimental import pallas as pl
from jax.experimental.pallas import tpu as pltpu
from jax.experimental.pallas import tpu_sc as plsc

_NJ = 200
_IB = 128
_D = 64


def _body(num_cores, tok_hbm, w_hbm, out_hbm, idx_v, rows_v, stage_v, sem_i, sem_g, sem_o):
    wid = lax.axis_index("s") * num_cores + lax.axis_index("c")
    i_base = wid * _IB

    def idx_start(b, bslot):
        pltpu.async_copy(
            tok_hbm.at[pl.ds(b * 8, 8), pl.ds(i_base, _IB)], idx_v.at[bslot], sem_i
        )

    def idx_wait():
        pltpu.make_async_copy(
            tok_hbm.at[pl.ds(0, 8), pl.ds(0, _IB)], idx_v.at[0], sem_i
        ).wait()

    def gather_start(j):
        rs = lax.rem(j, 3) if not isinstance(j, int) else j % 3
        b = lax.div(j, 8) if not isinstance(j, int) else j // 8
        bslot = lax.rem(b, 2) if not isinstance(j, int) else b % 2
        r = lax.rem(j, 8) if not isinstance(j, int) else j % 8
        pltpu.async_copy(
            w_hbm.at[idx_v.at[bslot, r]],
            rows_v.at[pl.ds(rs * _IB, _IB)],
            sem_g,
        )

    def gather_wait(rs):
        pltpu.make_async_copy(
            w_hbm.at[pl.ds(0, _IB)],
            rows_v.at[pl.ds(rs * _IB, _IB)],
            sem_g,
        ).wait()

    def out_start(j, ss):
        pltpu.async_copy(
            stage_v.at[pl.ds(ss * _D, _D)],
            out_hbm.at[j, :, pl.ds(i_base, _IB)],
            sem_o,
        )

    def out_wait(ss):
        pltpu.make_async_copy(
            stage_v.at[pl.ds(ss * _D, _D)],
            out_hbm.at[0, :, pl.ds(i_base, _IB)],
            sem_o,
        ).wait()

    def transpose(rs, ss):
        base = jax.lax.iota(jnp.int32, 16) + rs * _IB
        rvecs = [base + 16 * c for c in range(8)]
        for c in range(8):
            for f0 in range(0, _D, 16):
                vals = [
                    plsc.load_gather(
                        rows_v, [rvecs[c], jnp.full((16,), f0 + k, jnp.int32)]
                    )
                    for k in range(16)
                ]
                for k in range(16):
                    stage_v[ss * _D + f0 + k, pl.ds(16 * c, 16)] = vals[k]

    idx_start(0, 0)
    idx_wait()
    gather_start(0)
    gather_start(1)

    def loop_body(j, carry):
        rs = lax.rem(j, 3)
        ss = lax.rem(j, 2)
        gather_wait(rs)

        @pl.when(lax.rem(j + 2, 8) == 0)
        def _():
            idx_wait()

        gather_start(j + 2)

        @pl.when(jnp.logical_and(lax.rem(j + 4, 8) == 0, j + 4 < _NJ))
        def _():
            bb = lax.div(j + 4, 8)
            idx_start(bb, lax.rem(bb, 2))

        @pl.when(j >= 2)
        def _():
            out_wait(ss)

        transpose(rs, ss)
        out_start(j, ss)
        return carry

    lax.fori_loop(0, _NJ - 2, loop_body, 0)

    for j in (_NJ - 2, _NJ - 1):
        rs, ss = j % 3, j % 2
        gather_wait(rs)
        out_wait(ss)
        transpose(rs, ss)
        out_start(j, ss)
    out_wait((_NJ - 2) % 2)
    out_wait((_NJ - 1) % 2)


def kernel(token_ids, W):
    B, T = token_ids.shape
    D = W.shape[1]
    tok_t = token_ids.T
    w_pad = jnp.pad(W, ((0, 0), (0, 128 - D)))
    info = plsc.get_sparse_core_info()

    mesh = plsc.VectorSubcoreMesh(core_axis_name="c", subcore_axis_name="s")
    kfn = pl.kernel(
        functools.partial(_body, info.num_cores),
        out_type=jax.ShapeDtypeStruct((T, D, B), jnp.float32),
        mesh=mesh,
        scratch_types=[
            pltpu.VMEM((2, 8, _IB), jnp.int32),
            pltpu.VMEM((3 * _IB, 128), jnp.float32),
            pltpu.VMEM((2 * _D, 128), jnp.float32),
            pltpu.SemaphoreType.DMA,
            pltpu.SemaphoreType.DMA,
            pltpu.SemaphoreType.DMA,
        ],
        compiler_params=pltpu.CompilerParams(needs_layout_passes=False),
    )
    out3 = kfn(tok_t, w_pad)
    return jnp.transpose(out3, (2, 0, 1))


# speedup vs baseline: 1.4258x; 1.0021x over previous
import functools

import jax
import jax.numpy as jnp
from jax import lax
from jax.experimental import pallas as pl
from jax.experimental.pallas import tpu as pltpu
from jax.experimental.pallas import tpu_sc as plsc

_NJ = 200
_IB = 128
_D = 64


def _body(num_cores, tok_hbm, w_hbm, out_hbm, idx_v, rows_v, stage_v, sem_i, sem_g, sem_o):
    wid = lax.axis_index("s") * num_cores + lax.axis_index("c")
    i_base = wid * _IB

    def idx_start(b, bslot):
        pltpu.async_copy(
            tok_hbm.at[pl.ds(b * 8, 8), pl.ds(i_base, _IB)], idx_v.at[bslot], sem_i
        )

    def idx_wait():
        pltpu.make_async_copy(
            tok_hbm.at[pl.ds(0, 8), pl.ds(0, _IB)], idx_v.at[0], sem_i
        ).wait()

    def gather_start(j):
        rs = lax.rem(j, 4) if not isinstance(j, int) else j % 4
        b = lax.div(j, 8) if not isinstance(j, int) else j // 8
        bslot = lax.rem(b, 2) if not isinstance(j, int) else b % 2
        r = lax.rem(j, 8) if not isinstance(j, int) else j % 8
        for h in range(2):
            pltpu.async_copy(
                w_hbm.at[idx_v.at[bslot, r, pl.ds(h * 64, 64)]],
                rows_v.at[pl.ds(rs * _IB + h * 64, 64)],
                sem_g,
            )

    def gather_wait(rs):
        for h in range(2):
            pltpu.make_async_copy(
                w_hbm.at[pl.ds(0, 64)],
                rows_v.at[pl.ds(rs * _IB + h * 64, 64)],
                sem_g,
            ).wait()

    def out_start(j, ss):
        pltpu.async_copy(
            stage_v.at[pl.ds(ss * _D, _D)],
            out_hbm.at[j, :, pl.ds(i_base, _IB)],
            sem_o,
        )

    def out_wait(ss):
        pltpu.make_async_copy(
            stage_v.at[pl.ds(ss * _D, _D)],
            out_hbm.at[0, :, pl.ds(i_base, _IB)],
            sem_o,
        ).wait()

    def transpose(rs, ss):
        base = jax.lax.iota(jnp.int32, 16) + rs * _IB
        rvecs = [base + 16 * c for c in range(8)]
        for c in range(8):
            for f0 in range(0, _D, 16):
                vals = [
                    plsc.load_gather(
                        rows_v, [rvecs[c], jnp.full((16,), f0 + k, jnp.int32)]
                    )
                    for k in range(16)
                ]
                for k in range(16):
                    stage_v[ss * _D + f0 + k, pl.ds(16 * c, 16)] = vals[k]

    idx_start(0, 0)
    idx_wait()
    gather_start(0)
    gather_start(1)
    gather_start(2)

    def loop_body(j, carry):
        rs = lax.rem(j, 4)
        ss = lax.rem(j, 2)
        gather_wait(rs)

        @pl.when(lax.rem(j + 3, 8) == 0)
        def _():
            idx_wait()

        gather_start(j + 3)

        @pl.when(jnp.logical_and(lax.rem(j + 5, 8) == 0, j + 5 < _NJ))
        def _():
            bb = lax.div(j + 5, 8)
            idx_start(bb, lax.rem(bb, 2))

        @pl.when(j >= 2)
        def _():
            out_wait(ss)

        transpose(rs, ss)
        out_start(j, ss)
        return carry

    lax.fori_loop(0, _NJ - 3, loop_body, 0)

    for j in (_NJ - 3, _NJ - 2, _NJ - 1):
        rs, ss = j % 4, j % 2
        gather_wait(rs)
        out_wait(ss)
        transpose(rs, ss)
        out_start(j, ss)
    out_wait((_NJ - 2) % 2)
    out_wait((_NJ - 1) % 2)


def kernel(token_ids, W):
    B, T = token_ids.shape
    D = W.shape[1]
    tok_t = token_ids.T
    w_pad = jnp.pad(W, ((0, 0), (0, 128 - D)))
    info = plsc.get_sparse_core_info()

    mesh = plsc.VectorSubcoreMesh(core_axis_name="c", subcore_axis_name="s")
    kfn = pl.kernel(
        functools.partial(_body, info.num_cores),
        out_type=jax.ShapeDtypeStruct((T, D, B), jnp.float32),
        mesh=mesh,
        scratch_types=[
            pltpu.VMEM((2, 8, _IB), jnp.int32),
            pltpu.VMEM((4 * _IB, 128), jnp.float32),
            pltpu.VMEM((2 * _D, 128), jnp.float32),
            pltpu.SemaphoreType.DMA,
            pltpu.SemaphoreType.DMA,
            pltpu.SemaphoreType.DMA,
        ],
        compiler_params=pltpu.CompilerParams(needs_layout_passes=False),
    )
    out3 = kfn(tok_t, w_pad)
    return jnp.transpose(out3, (2, 0, 1))


# speedup vs baseline: 2.3378x; 1.6397x over previous
import functools

import jax
import jax.numpy as jnp
from jax import lax
from jax.experimental import pallas as pl
from jax.experimental.pallas import tpu as pltpu
from jax.experimental.pallas import tpu_sc as plsc

_NJ = 200
_IB = 128
_D = 64


def _body(num_cores, tok_hbm, w_hbm, out_hbm, idx_v, rows_v, stage_v, sem_i, sem_g, sem_o):
    wid = lax.axis_index("s") * num_cores + lax.axis_index("c")
    i_base = wid * _IB

    def idx_start(b, bslot):
        pltpu.async_copy(
            tok_hbm.at[pl.ds(b * 8, 8), pl.ds(i_base, _IB)], idx_v.at[bslot], sem_i
        )

    def idx_wait():
        pltpu.make_async_copy(
            tok_hbm.at[pl.ds(0, 8), pl.ds(0, _IB)], idx_v.at[0], sem_i
        ).wait()

    def gather_start(j):
        rs = lax.rem(j, 4) if not isinstance(j, int) else j % 4
        b = lax.div(j, 8) if not isinstance(j, int) else j // 8
        bslot = lax.rem(b, 2) if not isinstance(j, int) else b % 2
        r = lax.rem(j, 8) if not isinstance(j, int) else j % 8
        for h in range(2):
            pltpu.async_copy(
                w_hbm.at[idx_v.at[bslot, r, pl.ds(h * 64, 64)]],
                rows_v.at[pl.ds(rs * _IB + h * 64, 64)],
                sem_g,
            )

    def gather_wait(rs):
        for h in range(2):
            pltpu.make_async_copy(
                w_hbm.at[pl.ds(0, 64)],
                rows_v.at[pl.ds(rs * _IB + h * 64, 64)],
                sem_g,
            ).wait()

    def out_start(j, ss):
        pltpu.async_copy(
            stage_v.at[pl.ds(ss * _D, _D)],
            out_hbm.at[j, :, pl.ds(i_base, _IB)],
            sem_o,
        )

    def out_wait(ss):
        pltpu.make_async_copy(
            stage_v.at[pl.ds(ss * _D, _D)],
            out_hbm.at[0, :, pl.ds(i_base, _IB)],
            sem_o,
        ).wait()

    def transpose(rs, ss):
        base = jax.lax.iota(jnp.int32, 16) + rs * _IB

        @functools.partial(plsc.parallel_loop, 0, 32, unroll=2)
        def _tloop(i):
            c16 = lax.rem(i, 8) * 16
            f0 = lax.div(i, 8) * 16
            rvec = base + c16
            vals = [
                plsc.load_gather(
                    rows_v, [rvec, jnp.full((16,), k, jnp.int32) + f0]
                )
                for k in range(16)
            ]
            for k in range(16):
                stage_v[ss * _D + f0 + k, pl.ds(c16, 16)] = vals[k]

    idx_start(0, 0)
    idx_wait()
    gather_start(0)
    gather_start(1)
    gather_start(2)

    def loop_body(j, carry):
        rs = lax.rem(j, 4)
        ss = lax.rem(j, 2)
        gather_wait(rs)

        @pl.when(lax.rem(j + 3, 8) == 0)
        def _():
            idx_wait()

        gather_start(j + 3)

        @pl.when(jnp.logical_and(lax.rem(j + 5, 8) == 0, j + 5 < _NJ))
        def _():
            bb = lax.div(j + 5, 8)
            idx_start(bb, lax.rem(bb, 2))

        @pl.when(j >= 2)
        def _():
            out_wait(ss)

        transpose(rs, ss)
        out_start(j, ss)
        return carry

    lax.fori_loop(0, _NJ - 3, loop_body, 0)

    for j in (_NJ - 3, _NJ - 2, _NJ - 1):
        rs, ss = j % 4, j % 2
        gather_wait(rs)
        out_wait(ss)
        transpose(rs, ss)
        out_start(j, ss)
    out_wait((_NJ - 2) % 2)
    out_wait((_NJ - 1) % 2)


def kernel(token_ids, W):
    B, T = token_ids.shape
    D = W.shape[1]
    tok_t = token_ids.T
    w_pad = jnp.pad(W, ((0, 0), (0, 128 - D)))
    info = plsc.get_sparse_core_info()

    mesh = plsc.VectorSubcoreMesh(core_axis_name="c", subcore_axis_name="s")
    kfn = pl.kernel(
        functools.partial(_body, info.num_cores),
        out_type=jax.ShapeDtypeStruct((T, D, B), jnp.float32),
        mesh=mesh,
        scratch_types=[
            pltpu.VMEM((2, 8, _IB), jnp.int32),
            pltpu.VMEM((4 * _IB, 128), jnp.float32),
            pltpu.VMEM((2 * _D, 128), jnp.float32),
            pltpu.SemaphoreType.DMA,
            pltpu.SemaphoreType.DMA,
            pltpu.SemaphoreType.DMA,
        ],
        compiler_params=pltpu.CompilerParams(needs_layout_passes=False),
    )
    out3 = kfn(tok_t, w_pad)
    return jnp.transpose(out3, (2, 0, 1))
